# baseline (speedup 1.0000x reference)
import jax
import jax.numpy as jnp
from jax.experimental import pallas as pl
from jax.experimental.pallas import tpu as pltpu

W_PRIMAL = 0.1
W_DUAL = 0.1
W_STAT = 0.6
W_COMP = 0.2


def _kkt_kernel(a_ref, x_ref, lam_ref, b_ref, c_ref, out_ref, acc_ref, *, nb, m, n, batch):
    b_i = pl.program_id(0)
    j = pl.program_id(1)

    a = a_ref[0]
    x2 = x_ref[0]
    lam2 = lam_ref[0]

    ax = jax.lax.dot_general(a, x2, (((1,), (1,)), ((), ())),
                             preferred_element_type=jnp.float32)
    axmb = ax - b_ref[0].T

    relu_axmb = jnp.maximum(axmb, 0.0)
    primal_p = jnp.sum(relu_axmb * relu_axmb)
    lam_col = lam2.T
    comp_t = lam_col * axmb
    comp_p = jnp.sum(comp_t * comp_t)
    relu_neg_lam = jnp.maximum(-lam2, 0.0)
    dual_p = jnp.sum(relu_neg_lam * relu_neg_lam)

    stat_part = jax.lax.dot_general(lam2, a, (((1,), (0,)), ((), ())),
                                    preferred_element_type=jnp.float32)

    @pl.when(j == 0)
    def _():
        acc_ref[...] = stat_part

    @pl.when(j != 0)
    def _():
        acc_ref[...] += stat_part

    @pl.when(jnp.logical_and(b_i == 0, j == 0))
    def _():
        out_ref[...] = jnp.zeros_like(out_ref)

    inv_mb = 1.0 / (m * batch)
    contrib = (W_PRIMAL * primal_p + W_DUAL * dual_p + W_COMP * comp_p) * inv_mb
    out_ref[...] += contrib.reshape(1, 1)

    @pl.when(j == nb - 1)
    def _():
        station = acc_ref[...] + c_ref[0]
        stat_p = jnp.sum(station * station)
        out_ref[...] += (W_STAT * stat_p / (n * batch)).reshape(1, 1)


def kernel(x_hat, lam_hat, A_list, b_pad, c_pad, b_mask, c_mask, m_sizes, n_sizes):
    batch, m, n = A_list.shape
    x = x_hat.reshape(batch, 1, n)
    lam = lam_hat.reshape(batch, 1, m)
    b3 = b_pad.reshape(batch, 1, m)
    c3 = c_pad.reshape(batch, 1, n)

    tm = 256
    nb = m // tm

    import functools
    body = functools.partial(_kkt_kernel, nb=nb, m=m, n=n, batch=batch)

    out = pl.pallas_call(
        body,
        grid=(batch, nb),
        in_specs=[
            pl.BlockSpec((1, tm, n), lambda b, j: (b, j, 0)),
            pl.BlockSpec((1, 1, n), lambda b, j: (b, 0, 0)),
            pl.BlockSpec((1, 1, tm), lambda b, j: (b, 0, j)),
            pl.BlockSpec((1, 1, tm), lambda b, j: (b, 0, j)),
            pl.BlockSpec((1, 1, n), lambda b, j: (b, 0, 0)),
        ],
        out_specs=pl.BlockSpec((1, 1), lambda b, j: (0, 0)),
        out_shape=jax.ShapeDtypeStruct((1, 1), jnp.float32),
        scratch_shapes=[pltpu.VMEM((1, n), jnp.float32)],
    )(A_list, x, lam, b3, c3)
    return out[0, 0]


# speedup vs baseline: 1.4361x; 1.4361x over previous
import jax
import jax.numpy as jnp
from jax.experimental import pallas as pl
from jax.experimental.pallas import tpu as pltpu

W_PRIMAL = 0.1
W_DUAL = 0.1
W_STAT = 0.6
W_COMP = 0.2


def _kkt_kernel(a_ref, x_ref, lam_ref, b_ref, c_ref, out_ref, acc_ref, *, nb, m, n, batch):
    b_i = pl.program_id(0)
    j = pl.program_id(1)

    a = a_ref[0]
    x2 = x_ref[0]
    lam2 = lam_ref[0]

    ax = jax.lax.dot_general(a, x2, (((1,), (1,)), ((), ())),
                             preferred_element_type=jnp.float32)
    axmb = ax - b_ref[0].T

    relu_axmb = jnp.maximum(axmb, 0.0)
    primal_p = jnp.sum(relu_axmb * relu_axmb)
    lam_col = lam2.T
    comp_t = lam_col * axmb
    comp_p = jnp.sum(comp_t * comp_t)
    relu_neg_lam = jnp.maximum(-lam2, 0.0)
    dual_p = jnp.sum(relu_neg_lam * relu_neg_lam)

    stat_part = jax.lax.dot_general(lam2, a, (((1,), (0,)), ((), ())),
                                    preferred_element_type=jnp.float32)

    @pl.when(j == 0)
    def _():
        acc_ref[...] = stat_part

    @pl.when(j != 0)
    def _():
        acc_ref[...] += stat_part

    @pl.when(jnp.logical_and(b_i == 0, j == 0))
    def _():
        out_ref[...] = jnp.zeros_like(out_ref)

    inv_mb = 1.0 / (m * batch)
    contrib = (W_PRIMAL * primal_p + W_DUAL * dual_p + W_COMP * comp_p) * inv_mb
    out_ref[...] += contrib.reshape(1, 1)

    @pl.when(j == nb - 1)
    def _():
        station = acc_ref[...] + c_ref[0]
        stat_p = jnp.sum(station * station)
        out_ref[...] += (W_STAT * stat_p / (n * batch)).reshape(1, 1)


def kernel(x_hat, lam_hat, A_list, b_pad, c_pad, b_mask, c_mask, m_sizes, n_sizes):
    batch, m, n = A_list.shape
    x = x_hat.reshape(batch, 1, n)
    lam = lam_hat.reshape(batch, 1, m)
    b3 = b_pad.reshape(batch, 1, m)
    c3 = c_pad.reshape(batch, 1, n)

    tm = 512
    nb = m // tm

    import functools
    body = functools.partial(_kkt_kernel, nb=nb, m=m, n=n, batch=batch)

    out = pl.pallas_call(
        body,
        grid=(batch, nb),
        in_specs=[
            pl.BlockSpec((1, tm, n), lambda b, j: (b, j, 0)),
            pl.BlockSpec((1, 1, n), lambda b, j: (b, 0, 0)),
            pl.BlockSpec((1, 1, tm), lambda b, j: (b, 0, j)),
            pl.BlockSpec((1, 1, tm), lambda b, j: (b, 0, j)),
            pl.BlockSpec((1, 1, n), lambda b, j: (b, 0, 0)),
        ],
        out_specs=pl.BlockSpec((1, 1), lambda b, j: (0, 0)),
        out_shape=jax.ShapeDtypeStruct((1, 1), jnp.float32),
        scratch_shapes=[pltpu.VMEM((1, n), jnp.float32)],
    )(A_list, x, lam, b3, c3)
    return out[0, 0]


# speedup vs baseline: 1.7955x; 1.2502x over previous
import jax
import jax.numpy as jnp
from jax.experimental import pallas as pl
from jax.experimental.pallas import tpu as pltpu

W_PRIMAL = 0.1
W_DUAL = 0.1
W_STAT = 0.6
W_COMP = 0.2


def _kkt_kernel(a_ref, x_ref, lam_ref, b_ref, c_ref, out_ref, acc_ref, *, nb, m, n, batch):
    b_i = pl.program_id(0)
    j = pl.program_id(1)

    a = a_ref[0]
    x2 = x_ref[0]
    lam2 = lam_ref[0]

    ax = jax.lax.dot_general(a, x2, (((1,), (1,)), ((), ())),
                             preferred_element_type=jnp.float32)
    axmb = ax - b_ref[0].T

    relu_axmb = jnp.maximum(axmb, 0.0)
    primal_p = jnp.sum(relu_axmb * relu_axmb)
    lam_col = lam2.T
    comp_t = lam_col * axmb
    comp_p = jnp.sum(comp_t * comp_t)
    relu_neg_lam = jnp.maximum(-lam2, 0.0)
    dual_p = jnp.sum(relu_neg_lam * relu_neg_lam)

    stat_part = jax.lax.dot_general(lam2, a, (((1,), (0,)), ((), ())),
                                    preferred_element_type=jnp.float32)

    @pl.when(j == 0)
    def _():
        acc_ref[...] = stat_part

    @pl.when(j != 0)
    def _():
        acc_ref[...] += stat_part

    @pl.when(jnp.logical_and(b_i == 0, j == 0))
    def _():
        out_ref[...] = jnp.zeros_like(out_ref)

    inv_mb = 1.0 / (m * batch)
    contrib = (W_PRIMAL * primal_p + W_DUAL * dual_p + W_COMP * comp_p) * inv_mb
    out_ref[...] += contrib.reshape(1, 1)

    @pl.when(j == nb - 1)
    def _():
        station = acc_ref[...] + c_ref[0]
        stat_p = jnp.sum(station * station)
        out_ref[...] += (W_STAT * stat_p / (n * batch)).reshape(1, 1)


def kernel(x_hat, lam_hat, A_list, b_pad, c_pad, b_mask, c_mask, m_sizes, n_sizes):
    batch, m, n = A_list.shape
    x = x_hat.reshape(batch, 1, n)
    lam = lam_hat.reshape(batch, 1, m)
    b3 = b_pad.reshape(batch, 1, m)
    c3 = c_pad.reshape(batch, 1, n)

    tm = 1024
    nb = m // tm

    import functools
    body = functools.partial(_kkt_kernel, nb=nb, m=m, n=n, batch=batch)

    out = pl.pallas_call(
        body,
        grid=(batch, nb),
        in_specs=[
            pl.BlockSpec((1, tm, n), lambda b, j: (b, j, 0)),
            pl.BlockSpec((1, 1, n), lambda b, j: (b, 0, 0)),
            pl.BlockSpec((1, 1, tm), lambda b, j: (b, 0, j)),
            pl.BlockSpec((1, 1, tm), lambda b, j: (b, 0, j)),
            pl.BlockSpec((1, 1, n), lambda b, j: (b, 0, 0)),
        ],
        out_specs=pl.BlockSpec((1, 1), lambda b, j: (0, 0)),
        out_shape=jax.ShapeDtypeStruct((1, 1), jnp.float32),
        scratch_shapes=[pltpu.VMEM((1, n), jnp.float32)],
    )(A_list, x, lam, b3, c3)
    return out[0, 0]


# speedup vs baseline: 2.0231x; 1.1268x over previous
import functools

import jax
import jax.numpy as jnp
from jax.experimental import pallas as pl

W_PRIMAL = 0.1
W_DUAL = 0.1
W_STAT = 0.6
W_COMP = 0.2


def _kkt_kernel(a_ref, x_ref, lam_ref, b_ref, c_ref, out_ref, *, m, n, batch):
    step = pl.program_id(0)

    a = a_ref[...]
    x2 = x_ref[...]
    lam2 = lam_ref[...]

    ax = jax.lax.dot_general(a, x2, (((2,), (2,)), ((0,), (0,))),
                             preferred_element_type=jnp.float32)
    axmb = ax - b_ref[...].transpose(0, 2, 1)

    relu_axmb = jnp.maximum(axmb, 0.0)
    primal_p = jnp.sum(relu_axmb * relu_axmb)
    comp_t = lam2.transpose(0, 2, 1) * axmb
    comp_p = jnp.sum(comp_t * comp_t)
    relu_neg_lam = jnp.maximum(-lam2, 0.0)
    dual_p = jnp.sum(relu_neg_lam * relu_neg_lam)

    stat_part = jax.lax.dot_general(lam2, a, (((2,), (1,)), ((0,), (0,))),
                                    preferred_element_type=jnp.float32)
    station = stat_part + c_ref[...]
    stat_p = jnp.sum(station * station)

    contrib = ((W_PRIMAL * primal_p + W_DUAL * dual_p + W_COMP * comp_p) / (m * batch)
               + W_STAT * stat_p / (n * batch))

    @pl.when(step == 0)
    def _():
        out_ref[...] = jnp.zeros_like(out_ref)

    out_ref[...] += contrib.reshape(1, 1)


def kernel(x_hat, lam_hat, A_list, b_pad, c_pad, b_mask, c_mask, m_sizes, n_sizes):
    batch, m, n = A_list.shape
    x = x_hat.reshape(batch, 1, n)
    lam = lam_hat.reshape(batch, 1, m)
    b3 = b_pad.reshape(batch, 1, m)
    c3 = c_pad.reshape(batch, 1, n)

    bb = 2
    steps = batch // bb

    body = functools.partial(_kkt_kernel, m=m, n=n, batch=batch)

    out = pl.pallas_call(
        body,
        grid=(steps,),
        in_specs=[
            pl.BlockSpec((bb, m, n), lambda s: (s, 0, 0)),
            pl.BlockSpec((bb, 1, n), lambda s: (s, 0, 0)),
            pl.BlockSpec((bb, 1, m), lambda s: (s, 0, 0)),
            pl.BlockSpec((bb, 1, m), lambda s: (s, 0, 0)),
            pl.BlockSpec((bb, 1, n), lambda s: (s, 0, 0)),
        ],
        out_specs=pl.BlockSpec((1, 1), lambda s: (0, 0)),
        out_shape=jax.ShapeDtypeStruct((1, 1), jnp.float32),
    )(A_list, x, lam, b3, c3)
    return out[0, 0]
